# baseline (speedup 1.0000x reference)
import functools

import numpy as np
import jax
import jax.numpy as jnp
from jax import lax
from jax.experimental import pallas as pl
from jax.experimental.pallas import tpu as pltpu
from jax.experimental.pallas import tpu_sc as plsc

_H = 20
_WD = 20
_F = 32
_K = 8
_N = _H * _WD
_NC = 2
_NS = 16
_NW = _NC * _NS
_CELLS_PER_W = 16
_N_PAD = _NW * _CELLS_PER_W
_ROWS_PER_W = _CELLS_PER_W * _K
_LANES = 16

_MASK_NP = np.zeros((_H, _WD, _K), dtype=bool)
_MASK_NP[: _H - 1, : _WD - 1, : _K - 1] = True
_MASK = _MASK_NP.reshape(_N * _K)


def _mm_body(x_ref, w_ref, o_ref):
    o_ref[...] = jnp.dot(x_ref[...], w_ref[...],
                         preferred_element_type=jnp.float32)


def _project(x_flat, w):
    return pl.pallas_call(
        _mm_body,
        out_shape=jax.ShapeDtypeStruct((_N, _F), jnp.float32),
    )(x_flat, w)


def _sc_body(xw_hbm, ids_hbm, sims_hbm, out_hbm, idx_v, sims_v, rows_v,
             out_v, sem):
    wid = lax.axis_index("s") * _NC + lax.axis_index("c")
    rbase = wid * _ROWS_PER_W
    cbase = wid * _CELLS_PER_W
    pltpu.sync_copy(ids_hbm.at[pl.ds(rbase, _ROWS_PER_W)], idx_v)
    pltpu.sync_copy(sims_hbm.at[pl.ds(rbase, _ROWS_PER_W)], sims_v)
    pltpu.async_copy(xw_hbm.at[idx_v], rows_v, sem).wait()
    for c2 in range(0, _CELLS_PER_W, 2):
        sv = sims_v[pl.ds(c2 * _K, _LANES)]
        for half, c in ((0, c2), (_K, c2 + 1)):
            r0 = c * _K
            s = sv[half]
            acc_lo = s * rows_v[r0, pl.ds(0, _LANES)]
            acc_hi = s * rows_v[r0, pl.ds(_LANES, _LANES)]
            for k in range(1, _K):
                r = r0 + k
                s = sv[half + k]
                acc_lo = acc_lo + s * rows_v[r, pl.ds(0, _LANES)]
                acc_hi = acc_hi + s * rows_v[r, pl.ds(_LANES, _LANES)]
            out_v[c, pl.ds(0, _LANES)] = acc_lo
            out_v[c, pl.ds(_LANES, _LANES)] = acc_hi
    pltpu.sync_copy(out_v, out_hbm.at[pl.ds(cbase, _CELLS_PER_W)])


_sc_gather = functools.partial(
    pl.kernel,
    out_type=jax.ShapeDtypeStruct((_N_PAD, _F), jnp.float32),
    mesh=plsc.VectorSubcoreMesh(core_axis_name="c", subcore_axis_name="s",
                                num_cores=_NC, num_subcores=_NS),
    scratch_types=[
        pltpu.VMEM((_ROWS_PER_W,), jnp.int32),
        pltpu.VMEM((_ROWS_PER_W,), jnp.float32),
        pltpu.VMEM((_ROWS_PER_W, _F), jnp.float32),
        pltpu.VMEM((_CELLS_PER_W, _F), jnp.float32),
        pltpu.SemaphoreType.DMA,
    ],
    compiler_params=pltpu.CompilerParams(use_tc_tiling_on_sc=False),
)(_sc_body)


def kernel(x, W, nearest_neighbors):
    x_flat = x.reshape(_N, _F)
    ids = nearest_neighbors[..., 1].astype(jnp.int32).reshape(_N * _K)
    sims = nearest_neighbors[..., 2].reshape(_N * _K)
    sims = jnp.where(_MASK, sims, jnp.float32(0.0))
    pad = _N_PAD * _K - _N * _K
    ids = jnp.pad(ids, (0, pad))
    sims = jnp.pad(sims, (0, pad))
    xw = _project(x_flat, W)
    out = _sc_gather(xw, ids, sims)
    return out[:_N].reshape(_H, _WD, _F)


# speedup vs baseline: 1.2797x; 1.2797x over previous
import functools

import jax
import jax.numpy as jnp
from jax import lax
from jax.experimental import pallas as pl
from jax.experimental.pallas import tpu as pltpu
from jax.experimental.pallas import tpu_sc as plsc

_H = 20
_WD = 20
_F = 32
_K = 8
_N = _H * _WD
_NC = 2
_NS = 16
_CELLS_PER_W = 16
_ACTIVE_W = _N // _CELLS_PER_W
_ROWS_PER_W = _CELLS_PER_W * _K
_NN_PER_W = _ROWS_PER_W * 3
_LANES = 16


def _sc_body(x_hbm, nn_hbm, out_hbm, nnv, idx_v, rows_v, out_v, sem):
    wid = lax.axis_index("s") * _NC + lax.axis_index("c")

    @pl.when(wid < _ACTIVE_W)
    def _():
        pltpu.sync_copy(nn_hbm.at[pl.ds(wid * _NN_PER_W, _NN_PER_W)], nnv)
        lane = lax.iota(jnp.int32, _LANES)
        k_lane = lane & (_K - 1)
        kvalid = k_lane < _K - 1
        sim_regs = []
        for v in range(_ROWS_PER_W // _LANES):
            pos = lane * 3 + (v * 3 * _LANES + 1)
            idf = plsc.load_gather(nnv, [pos])
            sif = plsc.load_gather(nnv, [pos + 1])
            cell = wid * _CELLS_PER_W + 2 * v + (lane >> 3)
            ci = cell // _WD
            cj = cell % _WD
            valid = kvalid & (ci < _H - 1) & (cj < _WD - 1)
            sim_regs.append(jnp.where(valid, sif, jnp.float32(0.0)))
            idx_v[pl.ds(v * _LANES, _LANES)] = idf.astype(jnp.int32)
        pltpu.async_copy(x_hbm.at[idx_v], rows_v, sem).wait()
        for v in range(_ROWS_PER_W // _LANES):
            sv = sim_regs[v]
            for half, c in ((0, 2 * v), (_K, 2 * v + 1)):
                r0 = c * _K
                s = sv[half]
                acc_lo = s * rows_v[r0, pl.ds(0, _LANES)]
                acc_hi = s * rows_v[r0, pl.ds(_LANES, _LANES)]
                for k in range(1, _K):
                    r = r0 + k
                    s = sv[half + k]
                    acc_lo = acc_lo + s * rows_v[r, pl.ds(0, _LANES)]
                    acc_hi = acc_hi + s * rows_v[r, pl.ds(_LANES, _LANES)]
                out_v[c, pl.ds(0, _LANES)] = acc_lo
                out_v[c, pl.ds(_LANES, _LANES)] = acc_hi
        pltpu.sync_copy(out_v, out_hbm.at[pl.ds(wid * _CELLS_PER_W,
                                                _CELLS_PER_W)])


_sc_gather = functools.partial(
    pl.kernel,
    out_type=jax.ShapeDtypeStruct((_N, _F), jnp.float32),
    mesh=plsc.VectorSubcoreMesh(core_axis_name="c", subcore_axis_name="s",
                                num_cores=_NC, num_subcores=_NS),
    scratch_types=[
        pltpu.VMEM((_NN_PER_W,), jnp.float32),
        pltpu.VMEM((_ROWS_PER_W,), jnp.int32),
        pltpu.VMEM((_ROWS_PER_W, _F), jnp.float32),
        pltpu.VMEM((_CELLS_PER_W, _F), jnp.float32),
        pltpu.SemaphoreType.DMA,
    ],
    compiler_params=pltpu.CompilerParams(use_tc_tiling_on_sc=False,
                                         needs_layout_passes=False),
)(_sc_body)


def _mm_body(g_ref, w_ref, o_ref):
    o_ref[...] = jnp.dot(g_ref[...], w_ref[...],
                         preferred_element_type=jnp.float32)


def _project(g, w):
    return pl.pallas_call(
        _mm_body,
        out_shape=jax.ShapeDtypeStruct((_N, _F), jnp.float32),
    )(g, w)


def kernel(x, W, nearest_neighbors):
    x_flat = x.reshape(_N, _F)
    nn_flat = nearest_neighbors.reshape(_N * _K * 3)
    g = _sc_gather(x_flat, nn_flat)
    out = _project(g, W)
    return out.reshape(_H, _WD, _F)


# speedup vs baseline: 1.3471x; 1.0527x over previous
import functools

import jax
import jax.numpy as jnp
from jax import lax
from jax.experimental import pallas as pl
from jax.experimental.pallas import tpu as pltpu
from jax.experimental.pallas import tpu_sc as plsc

_H = 20
_WD = 20
_F = 32
_K = 8
_N = _H * _WD
_NC = 2
_NS = 16
_CELLS_PER_W = _WD
_ROWS_PER_W = _CELLS_PER_W * _K
_GATHER_SPLIT = 80
_LANES = 16


def _mm_body(x_ref, w_ref, xw_ref):
    w = w_ref[...]
    for i in range(_H):
        xw_ref[pl.ds(i * _WD, _WD), pl.ds(0, _F)] = jnp.dot(
            x_ref[i], w, preferred_element_type=jnp.float32)


def _project(x, w):
    return pl.pallas_call(
        _mm_body,
        out_shape=jax.ShapeDtypeStruct((_N, 128), jnp.float32),
    )(x, w)


def _sc_body(xw_hbm, nn_hbm, out_hbm, nnv, idx_v, rows_v, out_v, sem):
    wid = lax.axis_index("s") * _NC + lax.axis_index("c")

    @pl.when(wid < _H)
    def _():
        pltpu.sync_copy(nn_hbm.at[wid], nnv)
        lane = lax.iota(jnp.int32, _LANES)
        b_vec = lane & (_K - 1)
        half_cell = lane >> 3
        kvalid = b_vec < _K - 1
        sim_regs = []
        for v in range(_ROWS_PER_W // _LANES):
            a_vec = 2 * v + half_cell
            idf = plsc.load_gather(
                nnv, [a_vec, b_vec, jnp.full((_LANES,), 1, jnp.int32)])
            sif = plsc.load_gather(
                nnv, [a_vec, b_vec, jnp.full((_LANES,), 2, jnp.int32)])
            valid = kvalid & (a_vec < _WD - 1) & (wid < _H - 1)
            sim_regs.append(jnp.where(valid, sif, jnp.float32(0.0)))
            idx_v[pl.ds(v * _LANES, _LANES)] = idf.astype(jnp.int32)
        cp0 = pltpu.async_copy(
            xw_hbm.at[idx_v.at[pl.ds(0, _GATHER_SPLIT)]],
            rows_v.at[pl.ds(0, _GATHER_SPLIT)], sem)
        cp1 = pltpu.async_copy(
            xw_hbm.at[idx_v.at[pl.ds(_GATHER_SPLIT, _GATHER_SPLIT)]],
            rows_v.at[pl.ds(_GATHER_SPLIT, _GATHER_SPLIT)], sem)
        cp0.wait()
        cp1.wait()
        for v in range(_ROWS_PER_W // _LANES):
            sv = sim_regs[v]
            for half, c in ((0, 2 * v), (_K, 2 * v + 1)):
                r0 = c * _K
                s = sv[half]
                acc_lo = s * rows_v[r0, pl.ds(0, _LANES)]
                acc_hi = s * rows_v[r0, pl.ds(_LANES, _LANES)]
                for k in range(1, _K):
                    r = r0 + k
                    s = sv[half + k]
                    acc_lo = acc_lo + s * rows_v[r, pl.ds(0, _LANES)]
                    acc_hi = acc_hi + s * rows_v[r, pl.ds(_LANES, _LANES)]
                out_v[c, pl.ds(0, _LANES)] = acc_lo
                out_v[c, pl.ds(_LANES, _LANES)] = acc_hi
        pltpu.sync_copy(out_v, out_hbm.at[wid])


_sc_gather = functools.partial(
    pl.kernel,
    out_type=jax.ShapeDtypeStruct((_H, _WD, _F), jnp.float32),
    mesh=plsc.VectorSubcoreMesh(core_axis_name="c", subcore_axis_name="s",
                                num_cores=_NC, num_subcores=_NS),
    scratch_types=[
        pltpu.VMEM((_WD, _K, 3), jnp.float32),
        pltpu.VMEM((_ROWS_PER_W,), jnp.int32),
        pltpu.VMEM((_ROWS_PER_W, 128), jnp.float32),
        pltpu.VMEM((_CELLS_PER_W, _F), jnp.float32),
        pltpu.SemaphoreType.DMA,
    ],
    compiler_params=pltpu.CompilerParams(use_tc_tiling_on_sc=False,
                                         needs_layout_passes=False),
)(_sc_body)


def kernel(x, W, nearest_neighbors):
    xw = _project(x, W)
    return _sc_gather(xw, nearest_neighbors)
